# baseline (device time: 29052 ns/iter reference)
import jax
import jax.numpy as jnp
from jax import lax
from jax.experimental import pallas as pl
from jax.experimental.pallas import tpu as pltpu

N_DEV = 4


def _allgather_body(x_ref, d_ref, xout_ref, dout_ref,
                    xcomm, dcomm, xsend_sems, xrecv_sems,
                    dsend_sems, drecv_sems):
    me = lax.axis_index("i")
    left = (me - 1) % N_DEV
    right = (me + 1) % N_DEV

    barrier_sem = pltpu.get_barrier_semaphore()
    for nbr in (left, right):
        pl.semaphore_signal(
            barrier_sem, inc=1,
            device_id=(nbr,), device_id_type=pl.DeviceIdType.MESH,
        )
    pl.semaphore_wait(barrier_sem, 2)

    m_per = x_ref.shape[0]

    xout_ref[pl.ds(me * m_per, m_per), :] = x_ref[:, :]
    dout_ref[pl.ds(me, 1), :] = d_ref[:, :]
    xcomm[0, :, :] = x_ref[:, :]
    dcomm[0, :, :] = d_ref[:, :]

    for h in range(N_DEV - 1):
        rdma_x = pltpu.make_async_remote_copy(
            src_ref=xcomm.at[h],
            dst_ref=xcomm.at[h + 1],
            send_sem=xsend_sems.at[h],
            recv_sem=xrecv_sems.at[h],
            device_id=(right,),
            device_id_type=pl.DeviceIdType.MESH,
        )
        rdma_d = pltpu.make_async_remote_copy(
            src_ref=dcomm.at[h],
            dst_ref=dcomm.at[h + 1],
            send_sem=dsend_sems.at[h],
            recv_sem=drecv_sems.at[h],
            device_id=(right,),
            device_id_type=pl.DeviceIdType.MESH,
        )
        rdma_x.start()
        rdma_d.start()
        rdma_x.wait()
        rdma_d.wait()

        origin = (me - h - 1) % N_DEV
        xout_ref[pl.ds(origin * m_per, m_per), :] = xcomm[h + 1, :, :]
        dout_ref[pl.ds(origin, 1), :] = dcomm[h + 1, :, :]


def kernel(x, dest):
    m_per, n = x.shape
    x16 = x.astype(jnp.bfloat16)
    d2 = dest.reshape(1, m_per).astype(jnp.int32)

    x_all, d_all = pl.pallas_call(
        _allgather_body,
        out_shape=(
            jax.ShapeDtypeStruct((N_DEV * m_per, n), jnp.bfloat16),
            jax.ShapeDtypeStruct((N_DEV, m_per), jnp.int32),
        ),
        in_specs=[
            pl.BlockSpec(memory_space=pltpu.VMEM),
            pl.BlockSpec(memory_space=pltpu.VMEM),
        ],
        out_specs=(
            pl.BlockSpec(memory_space=pltpu.VMEM),
            pl.BlockSpec(memory_space=pltpu.VMEM),
        ),
        scratch_shapes=[
            pltpu.VMEM((N_DEV, m_per, n), jnp.bfloat16),
            pltpu.VMEM((N_DEV, 1, m_per), jnp.int32),
            pltpu.SemaphoreType.DMA((N_DEV - 1,)),
            pltpu.SemaphoreType.DMA((N_DEV - 1,)),
            pltpu.SemaphoreType.DMA((N_DEV - 1,)),
            pltpu.SemaphoreType.DMA((N_DEV - 1,)),
        ],
        compiler_params=pltpu.CompilerParams(collective_id=0),
    )(x16, d2)

    me = lax.axis_index("i")
    dest_all = d_all.reshape(N_DEV * m_per)
    mask = dest_all == me
    pos = jnp.cumsum(mask.astype(jnp.int32)) - 1
    scat = jnp.where(mask, pos, N_DEV * m_per)
    src_idx = (
        jnp.zeros((m_per,), jnp.int32)
        .at[scat]
        .set(jnp.arange(N_DEV * m_per, dtype=jnp.int32), mode="drop")
    )
    return x_all[src_idx]


# device time: 20144 ns/iter; 1.4422x vs baseline; 1.4422x over previous
import jax
import jax.numpy as jnp
from jax import lax
from jax.experimental import pallas as pl
from jax.experimental.pallas import tpu as pltpu

N_DEV = 4


def _body(x_ref, d_ref, out_ref,
          xall, dall, xcomm, dcomm,
          xsend_sems, xrecv_sems, dsend_sems, drecv_sems):
    me = lax.axis_index("i")
    left = (me - 1) % N_DEV
    right = (me + 1) % N_DEV

    m_per = x_ref.shape[0]
    n = x_ref.shape[1]

    barrier_sem = pltpu.get_barrier_semaphore()
    for nbr in (left, right):
        pl.semaphore_signal(
            barrier_sem, inc=1,
            device_id=(nbr,), device_id_type=pl.DeviceIdType.MESH,
        )
    pl.semaphore_wait(barrier_sem, 2)

    xall[pl.ds(me * m_per, m_per), :] = x_ref[:, :]
    dall[pl.ds(me, 1), :] = d_ref[:, :]
    xcomm[0, :, :] = x_ref[:, :]
    dcomm[0, :, :] = d_ref[:, :]

    for h in range(N_DEV - 1):
        rdma_x = pltpu.make_async_remote_copy(
            src_ref=xcomm.at[h], dst_ref=xcomm.at[h + 1],
            send_sem=xsend_sems.at[h], recv_sem=xrecv_sems.at[h],
            device_id=(right,), device_id_type=pl.DeviceIdType.MESH,
        )
        rdma_d = pltpu.make_async_remote_copy(
            src_ref=dcomm.at[h], dst_ref=dcomm.at[h + 1],
            send_sem=dsend_sems.at[h], recv_sem=drecv_sems.at[h],
            device_id=(right,), device_id_type=pl.DeviceIdType.MESH,
        )
        rdma_x.start()
        rdma_d.start()
        rdma_x.wait()
        rdma_d.wait()

        origin = (me - h - 1) % N_DEV
        xall[pl.ds(origin * m_per, m_per), :] = xcomm[h + 1, :, :]
        dall[pl.ds(origin, 1), :] = dcomm[h + 1, :, :]

    tri = (
        lax.broadcasted_iota(jnp.int32, (m_per, m_per), 0)
        <= lax.broadcasted_iota(jnp.int32, (m_per, m_per), 1)
    ).astype(jnp.bfloat16)
    row_iota = lax.broadcasted_iota(jnp.int32, (m_per, m_per), 0)

    acc = jnp.zeros((m_per, n), jnp.float32)
    off = jnp.float32(0.0)
    for s in range(N_DEV):
        dest_s = dall[pl.ds(s, 1), :]
        mb = dest_s == me
        mask16 = mb.astype(jnp.bfloat16)
        incl = jnp.dot(mask16, tri, preferred_element_type=jnp.float32)
        pos = (off + incl - 1.0).astype(jnp.int32)
        eq = jnp.logical_and(row_iota == pos, mb)
        block = xall[pl.ds(s * m_per, m_per), :]
        acc = acc + jnp.dot(
            eq.astype(jnp.bfloat16), block, preferred_element_type=jnp.float32
        )
        off = off + jnp.sum(mask16.astype(jnp.float32))

    out_ref[:, :] = acc.astype(jnp.bfloat16)


def kernel(x, dest):
    m_per, n = x.shape
    x16 = x.astype(jnp.bfloat16)
    d2 = dest.reshape(1, m_per).astype(jnp.int32)

    return pl.pallas_call(
        _body,
        out_shape=jax.ShapeDtypeStruct((m_per, n), jnp.bfloat16),
        in_specs=[
            pl.BlockSpec(memory_space=pltpu.VMEM),
            pl.BlockSpec(memory_space=pltpu.VMEM),
        ],
        out_specs=pl.BlockSpec(memory_space=pltpu.VMEM),
        scratch_shapes=[
            pltpu.VMEM((N_DEV * m_per, n), jnp.bfloat16),
            pltpu.VMEM((N_DEV, m_per), jnp.int32),
            pltpu.VMEM((N_DEV, m_per, n), jnp.bfloat16),
            pltpu.VMEM((N_DEV, 1, m_per), jnp.int32),
            pltpu.SemaphoreType.DMA((N_DEV - 1,)),
            pltpu.SemaphoreType.DMA((N_DEV - 1,)),
            pltpu.SemaphoreType.DMA((N_DEV - 1,)),
            pltpu.SemaphoreType.DMA((N_DEV - 1,)),
        ],
        compiler_params=pltpu.CompilerParams(collective_id=0),
    )(x16, d2)


# device time: 12672 ns/iter; 2.2926x vs baseline; 1.5896x over previous
import jax
import jax.numpy as jnp
from jax import lax
from jax.experimental import pallas as pl
from jax.experimental.pallas import tpu as pltpu

N_DEV = 4
N_RDMA = 7


def _body(x_ref, d_ref, out_ref,
          a_buf, b_buf, d_half, da, db, dd,
          send_sems, recv_sems):
    me = lax.axis_index("i")
    left = (me - 1) % N_DEV
    right = (me + 1) % N_DEV

    m_per = x_ref.shape[0]
    n = x_ref.shape[1]
    half = m_per // 2

    barrier_sem = pltpu.get_barrier_semaphore()
    for nbr in (left, right):
        pl.semaphore_signal(
            barrier_sem, inc=1,
            device_id=(nbr,), device_id_type=pl.DeviceIdType.MESH,
        )
    pl.semaphore_wait(barrier_sem, 2)

    def mk(idx, src, dst, dev):
        return pltpu.make_async_remote_copy(
            src_ref=src, dst_ref=dst,
            send_sem=send_sems.at[idx], recv_sem=recv_sems.at[idx],
            device_id=(dev,), device_id_type=pl.DeviceIdType.MESH,
        )

    r0 = mk(0, d_ref, da, right)
    r1 = mk(1, d_ref, db, left)
    r2 = mk(2, x_ref, a_buf, right)
    r3 = mk(3, x_ref, b_buf, left)
    r0.start()
    r1.start()
    r2.start()
    r3.start()

    r0.wait()
    r2.wait()
    r4 = mk(4, da, dd, right)
    r5 = mk(5, a_buf.at[pl.ds(0, half)], d_half.at[pl.ds(0, half)], right)
    r4.start()
    r5.start()

    r1.wait()
    r3.wait()
    r6 = mk(6, b_buf.at[pl.ds(half, half)], d_half.at[pl.ds(half, half)], left)
    r6.start()

    r4.wait()

    tri = (
        lax.broadcasted_iota(jnp.int32, (m_per, m_per), 0)
        <= lax.broadcasted_iota(jnp.int32, (m_per, m_per), 1)
    ).astype(jnp.bfloat16)
    row_iota = lax.broadcasted_iota(jnp.int32, (m_per, m_per), 0)

    dests = [d_ref[:, :], db[:, :], dd[:, :], da[:, :]]
    gids = [me, right, (me + 2) % N_DEV, left]
    masks = [d == me for d in dests]
    cnts = [jnp.sum(m.astype(jnp.float32)) for m in masks]
    offs = []
    for r in range(N_DEV):
        off = jnp.float32(0.0)
        for rp in range(N_DEV):
            if rp != r:
                off = off + jnp.where(gids[rp] < gids[r], cnts[rp], 0.0)
        offs.append(off)

    def one_hot(r):
        mb = masks[r]
        incl = jnp.dot(mb.astype(jnp.bfloat16), tri,
                       preferred_element_type=jnp.float32)
        pos = (offs[r] + incl - 1.0).astype(jnp.int32)
        return jnp.logical_and(row_iota == pos, mb).astype(jnp.bfloat16)

    acc = jnp.dot(one_hot(0), x_ref[:, :], preferred_element_type=jnp.float32)
    acc = acc + jnp.dot(one_hot(1), b_buf[:, :],
                        preferred_element_type=jnp.float32)
    acc = acc + jnp.dot(one_hot(3), a_buf[:, :],
                        preferred_element_type=jnp.float32)
    eq_d = one_hot(2)

    r5.wait()
    r6.wait()
    acc = acc + jnp.dot(eq_d, d_half[:, :], preferred_element_type=jnp.float32)

    out_ref[:, :] = acc.astype(jnp.bfloat16)


def kernel(x, dest):
    m_per, n = x.shape
    x16 = x.astype(jnp.bfloat16)
    d2 = dest.reshape(1, m_per).astype(jnp.int32)

    return pl.pallas_call(
        _body,
        out_shape=jax.ShapeDtypeStruct((m_per, n), jnp.bfloat16),
        in_specs=[
            pl.BlockSpec(memory_space=pltpu.VMEM),
            pl.BlockSpec(memory_space=pltpu.VMEM),
        ],
        out_specs=pl.BlockSpec(memory_space=pltpu.VMEM),
        scratch_shapes=[
            pltpu.VMEM((m_per, n), jnp.bfloat16),
            pltpu.VMEM((m_per, n), jnp.bfloat16),
            pltpu.VMEM((m_per, n), jnp.bfloat16),
            pltpu.VMEM((1, m_per), jnp.int32),
            pltpu.VMEM((1, m_per), jnp.int32),
            pltpu.VMEM((1, m_per), jnp.int32),
            pltpu.SemaphoreType.DMA((N_RDMA,)),
            pltpu.SemaphoreType.DMA((N_RDMA,)),
        ],
        compiler_params=pltpu.CompilerParams(collective_id=0),
    )(x16, d2)


# device time: 11199 ns/iter; 2.5942x vs baseline; 1.1315x over previous
import jax
import jax.numpy as jnp
from jax import lax
from jax.experimental import pallas as pl
from jax.experimental.pallas import tpu as pltpu

N_DEV = 4
N_RDMA = 9
K = 192


def _body(x_ref, d_ref, out_ref,
          pad_r_buf, pad_l_buf, pad_d_buf,
          fl_buf, fr_buf, diag_buf, sl_buf, sr_buf,
          da, db, dd,
          send_sems, recv_sems):
    me = lax.axis_index("i")
    left = (me - 1) % N_DEV
    right = (me + 1) % N_DEV

    m_per = x_ref.shape[0]
    half = K // 2

    barrier_sem = pltpu.get_barrier_semaphore()
    for nbr in (left, right):
        pl.semaphore_signal(
            barrier_sem, inc=1,
            device_id=(nbr,), device_id_type=pl.DeviceIdType.MESH,
        )
    pl.semaphore_wait(barrier_sem, 2)

    def mk(idx, src, dst, dev):
        return pltpu.make_async_remote_copy(
            src_ref=src, dst_ref=dst,
            send_sem=send_sems.at[idx], recv_sem=recv_sems.at[idx],
            device_id=(dev,), device_id_type=pl.DeviceIdType.MESH,
        )

    r0 = mk(0, d_ref, da, right)
    r1 = mk(1, d_ref, db, left)
    r0.start()
    r1.start()

    tri = (
        lax.broadcasted_iota(jnp.int32, (m_per, m_per), 0)
        <= lax.broadcasted_iota(jnp.int32, (m_per, m_per), 1)
    ).astype(jnp.bfloat16)
    k_iota = lax.broadcasted_iota(jnp.int32, (K, m_per), 0)

    def pack(dst_id):
        mb = d_ref[:, :] == dst_id
        incl = jnp.dot(mb.astype(jnp.bfloat16), tri,
                       preferred_element_type=jnp.float32)
        pos = (incl - 1.0).astype(jnp.int32)
        sel = jnp.logical_and(k_iota == pos, mb).astype(jnp.bfloat16)
        return jnp.dot(sel, x_ref[:, :],
                       preferred_element_type=jnp.float32).astype(jnp.bfloat16)

    pad_r_buf[:, :] = pack(right)
    pad_l_buf[:, :] = pack(left)
    pad_d_buf[:, :] = pack((me + 2) % N_DEV)

    r3 = mk(3, pad_r_buf, fl_buf, right)
    r4 = mk(4, pad_l_buf, fr_buf, left)
    r5 = mk(5, pad_d_buf.at[pl.ds(0, half)], sl_buf, right)
    r6 = mk(6, pad_d_buf.at[pl.ds(half, half)], sr_buf, left)
    r3.start()
    r4.start()
    r5.start()
    r6.start()

    r0.wait()
    r2 = mk(2, da, dd, right)
    r2.start()

    r5.wait()
    r7 = mk(7, sl_buf, diag_buf.at[pl.ds(0, half)], right)
    r7.start()
    r6.wait()
    r8 = mk(8, sr_buf, diag_buf.at[pl.ds(half, half)], left)
    r8.start()

    r1.wait()
    r2.wait()

    dests = [d_ref[:, :], db[:, :], dd[:, :], da[:, :]]
    gids = [me, right, (me + 2) % N_DEV, left]
    masks = [d == me for d in dests]
    cnts = [jnp.sum(m.astype(jnp.float32)) for m in masks]
    offs = []
    for r in range(N_DEV):
        off = jnp.float32(0.0)
        for rp in range(N_DEV):
            if rp != r:
                off = off + jnp.where(gids[rp] < gids[r], cnts[rp], 0.0)
        offs.append(off)

    row_iota = lax.broadcasted_iota(jnp.int32, (m_per, K), 0)
    col_iota = lax.broadcasted_iota(jnp.int32, (m_per, K), 1)

    def place(r):
        off_i = offs[r].astype(jnp.int32)
        cnt_i = cnts[r].astype(jnp.int32)
        q = jnp.logical_and(row_iota - col_iota == off_i, col_iota < cnt_i)
        return q.astype(jnp.bfloat16)

    own_pad_buf = pad_r_buf
    r3.wait_send()
    own_pad_buf[:, :] = pack(me)
    acc = jnp.dot(place(0), own_pad_buf[:, :],
                  preferred_element_type=jnp.float32)

    r4.wait()
    acc = acc + jnp.dot(place(1), fr_buf[:, :],
                        preferred_element_type=jnp.float32)
    r3.wait_recv()
    acc = acc + jnp.dot(place(3), fl_buf[:, :],
                        preferred_element_type=jnp.float32)

    q_diag = place(2)
    r7.wait()
    r8.wait()
    acc = acc + jnp.dot(q_diag, diag_buf[:, :],
                        preferred_element_type=jnp.float32)

    out_ref[:, :] = acc.astype(jnp.bfloat16)


def kernel(x, dest):
    m_per, n = x.shape
    x16 = x.astype(jnp.bfloat16)
    d2 = dest.reshape(1, m_per).astype(jnp.int32)

    return pl.pallas_call(
        _body,
        out_shape=jax.ShapeDtypeStruct((m_per, n), jnp.bfloat16),
        in_specs=[
            pl.BlockSpec(memory_space=pltpu.VMEM),
            pl.BlockSpec(memory_space=pltpu.VMEM),
        ],
        out_specs=pl.BlockSpec(memory_space=pltpu.VMEM),
        scratch_shapes=[
            pltpu.VMEM((K, n), jnp.bfloat16),
            pltpu.VMEM((K, n), jnp.bfloat16),
            pltpu.VMEM((K, n), jnp.bfloat16),
            pltpu.VMEM((K, n), jnp.bfloat16),
            pltpu.VMEM((K, n), jnp.bfloat16),
            pltpu.VMEM((K, n), jnp.bfloat16),
            pltpu.VMEM((K // 2, n), jnp.bfloat16),
            pltpu.VMEM((K // 2, n), jnp.bfloat16),
            pltpu.VMEM((1, m_per), jnp.int32),
            pltpu.VMEM((1, m_per), jnp.int32),
            pltpu.VMEM((1, m_per), jnp.int32),
            pltpu.SemaphoreType.DMA((N_RDMA,)),
            pltpu.SemaphoreType.DMA((N_RDMA,)),
        ],
        compiler_params=pltpu.CompilerParams(collective_id=0),
    )(x16, d2)


# device time: 10747 ns/iter; 2.7033x vs baseline; 1.0421x over previous
import jax
import jax.numpy as jnp
from jax import lax
from jax.experimental import pallas as pl
from jax.experimental.pallas import tpu as pltpu

N_DEV = 4
N_RDMA = 6
K = 192


def _body(x_ref, d_ref, out_ref,
          pad_r_buf, pad_l_buf, pad_d_buf,
          fl_buf, fr_buf, diag_buf,
          da, db, dd,
          send_sems, recv_sems):
    me = lax.axis_index("i")
    left = (me - 1) % N_DEV
    right = (me + 1) % N_DEV
    diag = (me + 2) % N_DEV

    m_per = x_ref.shape[0]

    barrier_sem = pltpu.get_barrier_semaphore()
    for nbr in (left, right, diag):
        pl.semaphore_signal(
            barrier_sem, inc=1,
            device_id=(nbr,), device_id_type=pl.DeviceIdType.MESH,
        )
    pl.semaphore_wait(barrier_sem, 3)

    def mk(idx, src, dst, dev):
        return pltpu.make_async_remote_copy(
            src_ref=src, dst_ref=dst,
            send_sem=send_sems.at[idx], recv_sem=recv_sems.at[idx],
            device_id=(dev,), device_id_type=pl.DeviceIdType.MESH,
        )

    r0 = mk(0, d_ref, da, right)
    r1 = mk(1, d_ref, db, left)
    r2 = mk(2, d_ref, dd, diag)
    r0.start()
    r1.start()
    r2.start()

    xv = x_ref[:, :].astype(jnp.bfloat16)
    tri = (
        lax.broadcasted_iota(jnp.int32, (m_per, m_per), 0)
        <= lax.broadcasted_iota(jnp.int32, (m_per, m_per), 1)
    ).astype(jnp.bfloat16)
    k_iota = lax.broadcasted_iota(jnp.int32, (K, m_per), 0)

    def pack(dst_id):
        mb = d_ref[:, :] == dst_id
        incl = jnp.dot(mb.astype(jnp.bfloat16), tri,
                       preferred_element_type=jnp.float32)
        pos = (incl - 1.0).astype(jnp.int32)
        sel = jnp.logical_and(k_iota == pos, mb).astype(jnp.bfloat16)
        return jnp.dot(sel, xv,
                       preferred_element_type=jnp.float32).astype(jnp.bfloat16)

    pad_d_buf[:, :] = pack(diag)
    r5 = mk(5, pad_d_buf, diag_buf, diag)
    r5.start()
    pad_r_buf[:, :] = pack(right)
    r3 = mk(3, pad_r_buf, fl_buf, right)
    r3.start()
    pad_l_buf[:, :] = pack(left)
    r4 = mk(4, pad_l_buf, fr_buf, left)
    r4.start()

    r0.wait()
    r1.wait()
    r2.wait()

    dests = [d_ref[:, :], db[:, :], dd[:, :], da[:, :]]
    gids = [me, right, diag, left]
    masks = [d == me for d in dests]
    cnts = [jnp.sum(m.astype(jnp.float32)) for m in masks]
    offs = []
    for r in range(N_DEV):
        off = jnp.float32(0.0)
        for rp in range(N_DEV):
            if rp != r:
                off = off + jnp.where(gids[rp] < gids[r], cnts[rp], 0.0)
        offs.append(off)

    row_iota = lax.broadcasted_iota(jnp.int32, (m_per, K), 0)
    col_iota = lax.broadcasted_iota(jnp.int32, (m_per, K), 1)

    def place(r):
        off_i = offs[r].astype(jnp.int32)
        cnt_i = cnts[r].astype(jnp.int32)
        q = jnp.logical_and(row_iota - col_iota == off_i, col_iota < cnt_i)
        return q.astype(jnp.bfloat16)

    acc = jnp.dot(place(0), pack(me), preferred_element_type=jnp.float32)

    r4.wait()
    acc = acc + jnp.dot(place(1), fr_buf[:, :],
                        preferred_element_type=jnp.float32)
    r3.wait()
    acc = acc + jnp.dot(place(3), fl_buf[:, :],
                        preferred_element_type=jnp.float32)
    q_diag = place(2)
    r5.wait()
    acc = acc + jnp.dot(q_diag, diag_buf[:, :],
                        preferred_element_type=jnp.float32)

    out_ref[:, :] = acc.astype(jnp.bfloat16)


def kernel(x, dest):
    m_per, n = x.shape
    d2 = dest.reshape(1, m_per).astype(jnp.int32)

    return pl.pallas_call(
        _body,
        out_shape=jax.ShapeDtypeStruct((m_per, n), jnp.bfloat16),
        in_specs=[
            pl.BlockSpec(memory_space=pltpu.VMEM),
            pl.BlockSpec(memory_space=pltpu.VMEM),
        ],
        out_specs=pl.BlockSpec(memory_space=pltpu.VMEM),
        scratch_shapes=[
            pltpu.VMEM((K, n), jnp.bfloat16),
            pltpu.VMEM((K, n), jnp.bfloat16),
            pltpu.VMEM((K, n), jnp.bfloat16),
            pltpu.VMEM((K, n), jnp.bfloat16),
            pltpu.VMEM((K, n), jnp.bfloat16),
            pltpu.VMEM((K, n), jnp.bfloat16),
            pltpu.VMEM((1, m_per), jnp.int32),
            pltpu.VMEM((1, m_per), jnp.int32),
            pltpu.VMEM((1, m_per), jnp.int32),
            pltpu.SemaphoreType.DMA((N_RDMA,)),
            pltpu.SemaphoreType.DMA((N_RDMA,)),
        ],
        compiler_params=pltpu.CompilerParams(collective_id=0),
    )(x, d2)
